# baseline (speedup 1.0000x reference)
import functools

import jax
import jax.numpy as jnp
from jax import lax
from jax.experimental import pallas as pl
from jax.experimental.pallas import tpu as pltpu
from jax.experimental.pallas import tpu_sc as plsc

NUM_TOKENS = 4096 * 200
DIM = 128
NUM_CORES = 2
NUM_SUBCORES = 16
NUM_WORKERS = NUM_CORES * NUM_SUBCORES
PER_WORKER = NUM_TOKENS // NUM_WORKERS
CHUNK = 128
NUM_CHUNKS = PER_WORKER // CHUNK
SUP = 256
GPS = SUP // CHUNK
NSUP = PER_WORKER // SUP
NBUF = 3

_mesh = plsc.VectorSubcoreMesh(core_axis_name="c", subcore_axis_name="s")


@functools.partial(
    pl.kernel,
    out_type=jax.ShapeDtypeStruct((NUM_TOKENS, DIM), jnp.float32),
    mesh=_mesh,
    scratch_types=[
        pltpu.VMEM((NUM_CHUNKS, CHUNK), jnp.int32),
        pltpu.VMEM((NBUF, SUP, DIM), jnp.float32),
        pltpu.SemaphoreType.DMA((NBUF,)),
        pltpu.SemaphoreType.DMA((NBUF,)),
    ],
)
def _gather_kernel(table_hbm, idx_hbm, out_hbm, idx_v, rows_v, gsem, ssem):
    wid = lax.axis_index("s") * NUM_CORES + lax.axis_index("c")
    base = wid * PER_WORKER

    pltpu.sync_copy(idx_hbm.at[wid], idx_v)

    def gathers_start(s, b):
        for k in range(GPS):
            pltpu.async_copy(
                table_hbm.at[idx_v.at[s * GPS + k]],
                rows_v.at[b, pl.ds(k * CHUNK, CHUNK)],
                gsem.at[b],
            )

    def gathers_wait(b):
        for _ in range(GPS):
            pltpu.make_async_copy(
                table_hbm.at[idx_v.at[0]],
                rows_v.at[0, pl.ds(0, CHUNK)],
                gsem.at[b],
            ).wait()

    def store_start(s, b):
        pltpu.async_copy(
            rows_v.at[b], out_hbm.at[pl.ds(base + s * SUP, SUP)], ssem.at[b]
        )

    def store_wait(b):
        pltpu.make_async_copy(
            rows_v.at[0], out_hbm.at[pl.ds(base, SUP)], ssem.at[b]
        ).wait()

    gathers_start(0, 0)
    gathers_start(1, 1)
    gathers_wait(0)
    store_start(0, 0)
    gathers_start(2, 2)

    def body(g, carry):
        s0 = NBUF * g + 1
        for k in range(NBUF):
            s = s0 + k
            b = (1 + k) % NBUF
            b2 = k % NBUF
            gathers_wait(b)
            store_start(s, b)
            store_wait(b2)
            gathers_start(s + 2, b2)
        return carry

    lax.fori_loop(0, (NSUP - 4) // NBUF, body, 0, unroll=False)

    s = NSUP - 3
    gathers_wait(s % NBUF)
    store_start(s, s % NBUF)
    store_wait((s + 2) % NBUF)
    gathers_start(s + 2, (s + 2) % NBUF)
    for s in (NSUP - 2, NSUP - 1):
        gathers_wait(s % NBUF)
        store_start(s, s % NBUF)
    for s in (NSUP - 3, NSUP - 2, NSUP - 1):
        store_wait(s % NBUF)


def kernel(token_ids, embedding_matrix):
    idx = token_ids.reshape(NUM_WORKERS, NUM_CHUNKS, CHUNK).astype(jnp.int32)
    out = _gather_kernel(embedding_matrix, idx)
    return out.reshape(token_ids.shape[0], token_ids.shape[1], DIM)


# speedup vs baseline: 1.0004x; 1.0004x over previous
import functools

import jax
import jax.numpy as jnp
from jax import lax
from jax.experimental import pallas as pl
from jax.experimental.pallas import tpu as pltpu
from jax.experimental.pallas import tpu_sc as plsc

NUM_TOKENS = 4096 * 200
DIM = 128
NUM_CORES = 2
NUM_SUBCORES = 16
NUM_WORKERS = NUM_CORES * NUM_SUBCORES
PER_WORKER = NUM_TOKENS // NUM_WORKERS
CHUNK = 128
NUM_CHUNKS = PER_WORKER // CHUNK
NBUF = 5
LAG_W = 2
LAG_S = 3
NUM_GROUPS = NUM_CHUNKS // NBUF

_mesh = plsc.VectorSubcoreMesh(core_axis_name="c", subcore_axis_name="s")


@functools.partial(
    pl.kernel,
    out_type=jax.ShapeDtypeStruct((NUM_TOKENS, DIM), jnp.float32),
    mesh=_mesh,
    scratch_types=[
        pltpu.VMEM((NUM_CHUNKS, CHUNK), jnp.int32),
        pltpu.VMEM((NBUF, CHUNK, DIM), jnp.float32),
        pltpu.SemaphoreType.DMA((NBUF,)),
        pltpu.SemaphoreType.DMA((NBUF,)),
    ],
)
def _gather_kernel(table_hbm, idx_hbm, out_hbm, idx_v, rows_v, gsem, ssem):
    wid = lax.axis_index("s") * NUM_CORES + lax.axis_index("c")
    base = wid * PER_WORKER

    pltpu.sync_copy(idx_hbm.at[wid], idx_v)

    def gather_start(j, b):
        pltpu.async_copy(table_hbm.at[idx_v.at[j]], rows_v.at[b], gsem.at[b])

    def gather_wait(b):
        pltpu.make_async_copy(
            table_hbm.at[idx_v.at[0]], rows_v.at[b], gsem.at[b]
        ).wait()

    def store_start(j, b):
        pltpu.async_copy(
            rows_v.at[b], out_hbm.at[pl.ds(base + j * CHUNK, CHUNK)], ssem.at[b]
        )

    def store_wait(b):
        pltpu.make_async_copy(
            rows_v.at[b], out_hbm.at[pl.ds(base, CHUNK)], ssem.at[b]
        ).wait()

    for t in range(NBUF):
        if t >= LAG_W:
            gather_wait(t - LAG_W)
        if t >= LAG_S:
            store_start(t - LAG_S, t - LAG_S)
        gather_start(t, t)

    def body(g, carry):
        t0 = g * NBUF
        for b in range(NBUF):
            t = t0 + b
            gather_wait((b - LAG_W) % NBUF)
            store_start(t - LAG_S, (b - LAG_S) % NBUF)
            store_wait(b)
            gather_start(t, b)
        return carry

    lax.fori_loop(1, NUM_GROUPS, body, 0, unroll=False)

    for t in range(NUM_CHUNKS, NUM_CHUNKS + LAG_W):
        gather_wait((t - LAG_W) % NBUF)
        store_start(t - LAG_S, (t - LAG_S) % NBUF)
    t = NUM_CHUNKS + LAG_S - 1
    store_start(t - LAG_S, (t - LAG_S) % NBUF)
    for b in range(NBUF):
        store_wait(b)


def kernel(token_ids, embedding_matrix):
    idx = token_ids.reshape(NUM_WORKERS, NUM_CHUNKS, CHUNK).astype(jnp.int32)
    out = _gather_kernel(embedding_matrix, idx)
    return out.reshape(token_ids.shape[0], token_ids.shape[1], DIM)
